# baseline (device time: 21014 ns/iter reference)
import jax
import jax.numpy as jnp
from jax import lax
from jax.experimental import pallas as pl
from jax.experimental.pallas import tpu as pltpu


def kernel(x, dy, gamma):
    m, d = x.shape

    def body(x_ref, dy_ref, out_ref, partial_ref, comm_ref, send_sem, recv_sem):
        my_x = lax.axis_index("x")
        my_y = lax.axis_index("y")
        nbr = (1 - my_x, my_y)

        xv = x_ref[:, :]
        dyv = dy_ref[:, :]
        mu = jnp.mean(xv, axis=1, keepdims=True)
        var = jnp.mean((xv - mu) * (xv - mu), axis=1, keepdims=True)
        rstd = lax.rsqrt(var + 1e-5)
        xhat = (xv - mu) * rstd
        partial_ref[0, :] = jnp.sum(dyv * xhat, axis=0)
        partial_ref[1, :] = jnp.sum(dyv, axis=0)

        barrier = pltpu.get_barrier_semaphore()
        pl.semaphore_signal(
            barrier, inc=1, device_id=nbr, device_id_type=pl.DeviceIdType.MESH
        )
        pl.semaphore_wait(barrier, 1)

        rdma = pltpu.make_async_remote_copy(
            src_ref=partial_ref,
            dst_ref=comm_ref,
            send_sem=send_sem,
            recv_sem=recv_sem,
            device_id=nbr,
            device_id_type=pl.DeviceIdType.MESH,
        )
        rdma.start()
        rdma.wait()

        out_ref[:, :] = partial_ref[:, :] + comm_ref[:, :]

    return pl.pallas_call(
        body,
        out_shape=jax.ShapeDtypeStruct((2, d), jnp.float32),
        in_specs=[
            pl.BlockSpec(memory_space=pltpu.VMEM),
            pl.BlockSpec(memory_space=pltpu.VMEM),
        ],
        out_specs=pl.BlockSpec(memory_space=pltpu.VMEM),
        scratch_shapes=[
            pltpu.VMEM((2, d), jnp.float32),
            pltpu.VMEM((2, d), jnp.float32),
            pltpu.SemaphoreType.DMA,
            pltpu.SemaphoreType.DMA,
        ],
        compiler_params=pltpu.CompilerParams(collective_id=0),
    )(x, dy)


# device time: 18509 ns/iter; 1.1353x vs baseline; 1.1353x over previous
import jax
import jax.numpy as jnp
from jax import lax
from jax.experimental import pallas as pl
from jax.experimental.pallas import tpu as pltpu

BLOCK_ROWS = 256
EPS = 1e-5


def kernel(x, dy, gamma):
    m, d = x.shape
    rows_per_dev = m // 2
    n_steps = rows_per_dev // BLOCK_ROWS

    def body(
        x_hbm,
        dy_hbm,
        out_ref,
        xbuf,
        dybuf,
        partial_ref,
        sum_ref,
        commx_ref,
        commy_ref,
        load_sems,
        sendx_sem,
        recvx_sem,
        sendy_sem,
        recvy_sem,
    ):
        my_x = lax.axis_index("x")
        my_y = lax.axis_index("y")
        x_nbr = (1 - my_x, my_y)
        y_nbr = (my_x, 1 - my_y)
        row0 = my_y * rows_per_dev

        def start_load(k, slot):
            cp_x = pltpu.make_async_copy(
                x_hbm.at[pl.ds(row0 + k * BLOCK_ROWS, BLOCK_ROWS), :],
                xbuf.at[slot],
                load_sems.at[slot, 0],
            )
            cp_dy = pltpu.make_async_copy(
                dy_hbm.at[pl.ds(row0 + k * BLOCK_ROWS, BLOCK_ROWS), :],
                dybuf.at[slot],
                load_sems.at[slot, 1],
            )
            cp_x.start()
            cp_dy.start()
            return cp_x, cp_dy

        start_load(0, 0)
        for k in range(n_steps):
            slot = k % 2
            if k + 1 < n_steps:
                start_load(k + 1, (k + 1) % 2)
            pltpu.make_async_copy(
                x_hbm.at[pl.ds(row0 + k * BLOCK_ROWS, BLOCK_ROWS), :],
                xbuf.at[slot],
                load_sems.at[slot, 0],
            ).wait()
            pltpu.make_async_copy(
                dy_hbm.at[pl.ds(row0 + k * BLOCK_ROWS, BLOCK_ROWS), :],
                dybuf.at[slot],
                load_sems.at[slot, 1],
            ).wait()

            xv = xbuf[slot]
            dyv = dybuf[slot]
            mu = jnp.mean(xv, axis=1, keepdims=True)
            xc = xv - mu
            var = jnp.mean(xc * xc, axis=1, keepdims=True)
            xhat = xc * lax.rsqrt(var + EPS)
            dgamma = jnp.sum(dyv * xhat, axis=0)
            dbeta = jnp.sum(dyv, axis=0)
            if k == 0:
                partial_ref[0, :] = dgamma
                partial_ref[1, :] = dbeta
            else:
                partial_ref[0, :] += dgamma
                partial_ref[1, :] += dbeta

        barrier = pltpu.get_barrier_semaphore()
        for nbr in (x_nbr, y_nbr):
            pl.semaphore_signal(
                barrier, inc=1, device_id=nbr, device_id_type=pl.DeviceIdType.MESH
            )
        pl.semaphore_wait(barrier, 2)

        rdma_x = pltpu.make_async_remote_copy(
            src_ref=partial_ref,
            dst_ref=commx_ref,
            send_sem=sendx_sem,
            recv_sem=recvx_sem,
            device_id=x_nbr,
            device_id_type=pl.DeviceIdType.MESH,
        )
        rdma_x.start()
        rdma_x.wait()
        sum_ref[:, :] = partial_ref[:, :] + commx_ref[:, :]

        rdma_y = pltpu.make_async_remote_copy(
            src_ref=sum_ref,
            dst_ref=commy_ref,
            send_sem=sendy_sem,
            recv_sem=recvy_sem,
            device_id=y_nbr,
            device_id_type=pl.DeviceIdType.MESH,
        )
        rdma_y.start()
        rdma_y.wait()
        out_ref[:, :] = sum_ref[:, :] + commy_ref[:, :]

    return pl.pallas_call(
        body,
        out_shape=jax.ShapeDtypeStruct((2, d), jnp.float32),
        in_specs=[
            pl.BlockSpec(memory_space=pl.ANY),
            pl.BlockSpec(memory_space=pl.ANY),
        ],
        out_specs=pl.BlockSpec(memory_space=pltpu.VMEM),
        scratch_shapes=[
            pltpu.VMEM((2, BLOCK_ROWS, d), jnp.float32),
            pltpu.VMEM((2, BLOCK_ROWS, d), jnp.float32),
            pltpu.VMEM((2, d), jnp.float32),
            pltpu.VMEM((2, d), jnp.float32),
            pltpu.VMEM((2, d), jnp.float32),
            pltpu.VMEM((2, d), jnp.float32),
            pltpu.SemaphoreType.DMA((2, 2)),
            pltpu.SemaphoreType.DMA,
            pltpu.SemaphoreType.DMA,
            pltpu.SemaphoreType.DMA,
            pltpu.SemaphoreType.DMA,
        ],
        compiler_params=pltpu.CompilerParams(collective_id=0),
    )(x, dy)
